# baseline (speedup 1.0000x reference)
import functools

import jax
import jax.numpy as jnp
from jax import lax
from jax.experimental import pallas as pl
from jax.experimental.pallas import tpu as pltpu
from jax.experimental.pallas import tpu_sc as plsc

N = 10000
DEG = 32
D = 128
ALPHA = 0.1
NITER = 10

NC, NS = 2, 16
NW = NC * NS
C = 128
E_TOT = N * DEG + N
SCH = 4
NSCH = 22
CPW = NSCH * SCH
EPW = CPW * C
E_PAD = EPW * NW
NSCH_STORE = NSCH + 2
NP = 10240
RPT = NP // NS
DUMMY = N + 16


def _sc_spmm(g, idx, zeros_blk):
    mesh = plsc.VectorSubcoreMesh(
        core_axis_name="c", subcore_axis_name="s",
        num_cores=NC, num_subcores=NS)

    @functools.partial(
        pl.kernel,
        out_type=jax.ShapeDtypeStruct((NC, NP, D), jnp.float32),
        mesh=mesh,
        scratch_types=[
            pltpu.VMEM((2 * SCH, C), jnp.int32),
            pltpu.VMEM((2 * SCH, C), jnp.int32),
            pltpu.VMEM((C, D), jnp.float32),
            pltpu.VMEM((C, D), jnp.float32),
            pltpu.VMEM_SHARED((NP, D), jnp.float32),
            pltpu.SemaphoreType.DMA,
            pltpu.SemaphoreType.DMA,
            pltpu.SemaphoreType.DMA,
            pltpu.SemaphoreType.DMA,
        ],
    )
    def k(g_hbm, idx_hbm, z_hbm, acc_hbm,
          ibufa, ibufb, gbuf0, gbuf1, accum, isema, isemb, gsem0, gsem1):
        cid = lax.axis_index("c")
        sid = lax.axis_index("s")
        wid = sid * NC + cid
        pltpu.sync_copy(z_hbm, accum.at[pl.ds(sid * RPT, RPT)])
        plsc.subcore_barrier()

        def icopy(s, ibuf, isem):
            return pltpu.make_async_copy(idx_hbm.at[wid, s], ibuf, isem)

        def gather(ibuf, k_, buf, sem):
            return pltpu.make_async_copy(g_hbm.at[ibuf.at[2 * k_]], buf, sem)

        def scat(ibuf, k_, buf):
            pltpu.sync_copy(buf, accum.at[ibuf.at[2 * k_ + 1]], add=True)

        icopy(0, ibufa, isema).start()
        icopy(1, ibufb, isemb).start()
        icopy(0, ibufa, isema).wait()
        gather(ibufa, 0, gbuf0, gsem0).start()
        gather(ibufa, 1, gbuf1, gsem1).start()

        def superchunk(s, cur, curs, nxt, nxts):
            gather(cur, 0, gbuf0, gsem0).wait()
            scat(cur, 0, gbuf0)
            gather(cur, 2, gbuf0, gsem0).start()
            gather(cur, 1, gbuf1, gsem1).wait()
            scat(cur, 1, gbuf1)
            gather(cur, 3, gbuf1, gsem1).start()
            gather(cur, 2, gbuf0, gsem0).wait()
            scat(cur, 2, gbuf0)
            icopy(s + 1, nxt, nxts).wait()
            gather(nxt, 0, gbuf0, gsem0).start()
            gather(cur, 3, gbuf1, gsem1).wait()
            scat(cur, 3, gbuf1)
            gather(nxt, 1, gbuf1, gsem1).start()
            icopy(s + 2, cur, curs).start()

        def pair(i, carry):
            s = 2 * i
            superchunk(s, ibufa, isema, ibufb, isemb)
            superchunk(s + 1, ibufb, isemb, ibufa, isema)
            return carry

        lax.fori_loop(0, NSCH // 2, pair, 0)
        icopy(NSCH + 1, ibufb, isemb).wait()
        gather(ibufa, 0, gbuf0, gsem0).wait()
        gather(ibufa, 1, gbuf1, gsem1).wait()
        plsc.subcore_barrier()
        pltpu.sync_copy(accum.at[pl.ds(sid * RPT, RPT)],
                        acc_hbm.at[cid, pl.ds(sid * RPT, RPT)])

    return k(g, idx, zeros_blk)


def _tc_combine(acc, w, b):
    BR = 256

    def body(a_ref, w_ref, b_ref, o_ref):
        o_ref[...] = w_ref[...] * (a_ref[0] + a_ref[1]) + b_ref[...]

    return pl.pallas_call(
        body,
        grid=(NP // BR,),
        in_specs=[
            pl.BlockSpec((NC, BR, D), lambda i: (0, i, 0)),
            pl.BlockSpec((BR, D), lambda i: (i, 0)),
            pl.BlockSpec((BR, D), lambda i: (i, 0)),
        ],
        out_specs=pl.BlockSpec((BR, D), lambda i: (i, 0)),
        out_shape=jax.ShapeDtypeStruct((NP, D), jnp.float32),
    )(acc, w, b)


def kernel(E, edge_index):
    loops = jnp.arange(N, dtype=edge_index.dtype)
    rows = jnp.concatenate([edge_index[0], loops])
    cols = jnp.concatenate([edge_index[1], loops])
    deg = jax.ops.segment_sum(
        jnp.ones((E_TOT,), jnp.float32), rows, num_segments=N)
    dinv = lax.rsqrt(deg)

    pad = E_PAD - E_TOT
    cols3 = jnp.concatenate(
        [cols, jnp.zeros((pad,), cols.dtype)]).reshape(NW, CPW, C)
    rows3 = jnp.concatenate(
        [rows, jnp.full((pad,), DUMMY, rows.dtype)]).reshape(NW, CPW, C)
    extra = (NSCH_STORE * SCH) - CPW
    cols3 = jnp.pad(cols3, ((0, 0), (0, extra), (0, 0)))
    rows3 = jnp.pad(rows3, ((0, 0), (0, extra), (0, 0)),
                    constant_values=DUMMY)
    idx = jnp.stack([cols3, rows3], axis=2).astype(jnp.int32)
    idx = idx.reshape(NW, NSCH_STORE, 2 * SCH, C)
    zeros_blk = jnp.zeros((RPT, D), jnp.float32)

    dcol = jnp.pad(dinv, (0, NP - N))[:, None]
    epad = jnp.pad(E, ((0, NP - N), (0, 0)))
    w2 = jnp.broadcast_to((1.0 - ALPHA) * dcol * dcol, (NP, D))
    w1 = jnp.broadcast_to((1.0 - ALPHA) * dcol, (NP, D))
    b2 = ALPHA * dcol * epad
    b1 = ALPHA * epad

    g = dcol * epad
    for it in range(NITER):
        acc = _sc_spmm(g, idx, zeros_blk)
        if it < NITER - 1:
            g = _tc_combine(acc, w2, b2)
        else:
            g = _tc_combine(acc, w1, b1)
    return g[:N]


# speedup vs baseline: 5.1011x; 5.1011x over previous
import functools

import jax
import jax.numpy as jnp
from jax import lax
from jax.experimental import pallas as pl
from jax.experimental.pallas import tpu as pltpu
from jax.experimental.pallas import tpu_sc as plsc

N = 10000
DEG = 32
D = 128
ALPHA = 0.1
NITER = 10

NC, NS = 2, 16
NW = NC * NS
C = 128
E_TOT = N * DEG + N
CPW = 81
EPW = CPW * C
E_PAD = EPW * NW
NP = 10240
RPT = NP // NS
DUMMY = N + 16


def _sc_spmm(g, cols3, rows3, zeros_blk):
    mesh = plsc.VectorSubcoreMesh(
        core_axis_name="c", subcore_axis_name="s",
        num_cores=NC, num_subcores=NS)

    @functools.partial(
        pl.kernel,
        out_type=jax.ShapeDtypeStruct((NC, NP, D), jnp.float32),
        mesh=mesh,
        scratch_types=[
            pltpu.VMEM((CPW, C), jnp.int32),
            pltpu.VMEM((CPW, C), jnp.int32),
            pltpu.VMEM((C, D), jnp.float32),
            pltpu.VMEM_SHARED((NP, D), jnp.float32),
            pltpu.SemaphoreType.DMA,
        ],
    )
    def k(g_hbm, c_hbm, r_hbm, z_hbm, acc_hbm,
          cbuf, rbuf, gbuf, accum, isem):
        cid = lax.axis_index("c")
        sid = lax.axis_index("s")
        wid = sid * NC + cid
        pltpu.make_async_copy(c_hbm.at[wid], cbuf, isem).start()
        pltpu.make_async_copy(r_hbm.at[wid], rbuf, isem).start()
        pltpu.sync_copy(z_hbm, accum.at[pl.ds(sid * RPT, RPT)])
        plsc.subcore_barrier()
        pltpu.make_async_copy(c_hbm.at[wid], cbuf, isem).wait()
        pltpu.make_async_copy(r_hbm.at[wid], rbuf, isem).wait()

        def chunk(k_, carry):
            pltpu.sync_copy(g_hbm.at[cbuf.at[k_]], gbuf)
            pltpu.sync_copy(gbuf, accum.at[rbuf.at[k_]], add=True)
            return carry

        lax.fori_loop(0, CPW, chunk, 0)
        plsc.subcore_barrier()
        pltpu.sync_copy(accum.at[pl.ds(sid * RPT, RPT)],
                        acc_hbm.at[cid, pl.ds(sid * RPT, RPT)])

    return k(g, cols3, rows3, zeros_blk)


def _tc_combine(acc, w, b):
    BR = 256

    def body(a_ref, w_ref, b_ref, o_ref):
        o_ref[...] = w_ref[...] * (a_ref[0] + a_ref[1]) + b_ref[...]

    return pl.pallas_call(
        body,
        grid=(NP // BR,),
        in_specs=[
            pl.BlockSpec((NC, BR, D), lambda i: (0, i, 0)),
            pl.BlockSpec((BR, D), lambda i: (i, 0)),
            pl.BlockSpec((BR, D), lambda i: (i, 0)),
        ],
        out_specs=pl.BlockSpec((BR, D), lambda i: (i, 0)),
        out_shape=jax.ShapeDtypeStruct((NP, D), jnp.float32),
    )(acc, w, b)


def kernel(E, edge_index):
    loops = jnp.arange(N, dtype=edge_index.dtype)
    rows = jnp.concatenate([edge_index[0], loops])
    cols = jnp.concatenate([edge_index[1], loops])
    deg = jax.ops.segment_sum(
        jnp.ones((E_TOT,), jnp.float32), rows, num_segments=N)
    dinv = lax.rsqrt(deg)

    pad = E_PAD - E_TOT
    cols3 = jnp.concatenate(
        [cols, jnp.zeros((pad,), cols.dtype)]).reshape(NW, CPW, C)
    rows3 = jnp.concatenate(
        [rows, jnp.full((pad,), DUMMY, rows.dtype)]).reshape(NW, CPW, C)
    cols3 = cols3.astype(jnp.int32)
    rows3 = rows3.astype(jnp.int32)
    zeros_blk = jnp.zeros((RPT, D), jnp.float32)

    dcol = jnp.pad(dinv, (0, NP - N))[:, None]
    epad = jnp.pad(E, ((0, NP - N), (0, 0)))
    w2 = jnp.broadcast_to((1.0 - ALPHA) * dcol * dcol, (NP, D))
    w1 = jnp.broadcast_to((1.0 - ALPHA) * dcol, (NP, D))
    b2 = ALPHA * dcol * epad
    b1 = ALPHA * epad

    g = dcol * epad
    for it in range(NITER):
        acc = _sc_spmm(g, cols3, rows3, zeros_blk)
        if it < NITER - 1:
            g = _tc_combine(acc, w2, b2)
        else:
            g = _tc_combine(acc, w1, b1)
    return g[:N]
